# baseline (speedup 1.0000x reference)
import functools

import jax
import jax.numpy as jnp
from jax import lax
from jax.experimental import pallas as pl
from jax.experimental.pallas import tpu as pltpu
from jax.experimental.pallas import tpu_sc as plsc

E = 8
K = 2
D = 2048
DFF = 8192
S = 2048

T = 768
NT = (K * S + E * (T - 1) + T - 1) // T
P = NT * T
BF = 512
NJ = DFF // BF
EP = 128
TR = 512

NC = 2
NS = 16
NW = NC * NS
L = 16


def _router_body(x_ref, wt_ref, b_ref, eo_ref, wo_ref):
    x = x_ref[...]
    logits = lax.dot_general(x, wt_ref[...], (((1,), (0,)), ((), ())),
                             preferred_element_type=jnp.float32)
    logits = logits + b_ref[0:1, :]
    lane = lax.broadcasted_iota(jnp.int32, (TR, EP), 1)
    m1 = jnp.max(logits, axis=1, keepdims=True)
    i1 = jnp.min(jnp.where(logits == m1, lane, EP), axis=1, keepdims=True)
    masked = jnp.where(lane == i1, -jnp.inf, logits)
    m2 = jnp.max(masked, axis=1, keepdims=True)
    i2 = jnp.min(jnp.where(masked == m2, lane, EP), axis=1, keepdims=True)
    w1 = 1.0 / (1.0 + jnp.exp(m2 - m1))
    eo_ref[...] = jnp.broadcast_to(i1 * E + i2, (TR, EP)).astype(jnp.int32)
    wo_ref[...] = jnp.broadcast_to(w1, (TR, EP))


def _run_router(xf, router_W, router_b):
    wt = jnp.zeros((D, EP), jnp.float32).at[:, :E].set(router_W.T)
    brow = jnp.full((EP,), -1e30, jnp.float32).at[:E].set(router_b)
    bp = jnp.broadcast_to(brow[None, :], (8, EP))
    eo, wo = pl.pallas_call(
        _router_body,
        grid=(S // TR,),
        in_specs=[
            pl.BlockSpec((TR, D), lambda r: (r, 0)),
            pl.BlockSpec((D, EP), lambda r: (0, 0)),
            pl.BlockSpec((8, EP), lambda r: (0, 0)),
        ],
        out_specs=[
            pl.BlockSpec((TR, EP), lambda r: (r, 0)),
            pl.BlockSpec((TR, EP), lambda r: (r, 0)),
        ],
        out_shape=[
            jax.ShapeDtypeStruct((S, EP), jnp.int32),
            jax.ShapeDtypeStruct((S, EP), jnp.float32),
        ],
    )(xf, wt, bp)
    e12 = eo[:, 0]
    w1 = wo[:, 0]
    return e12 // E, e12 % E, w1, 1.0 - w1


def _routing_metadata(e1, e2):
    i32 = jnp.int32
    e_all = jnp.concatenate([e1, e2]).astype(i32)
    onehot = (e_all[:, None] == jnp.arange(E, dtype=i32)[None, :]).astype(i32)
    ranks_inc = jnp.cumsum(onehot, axis=0)
    counts = ranks_inc[-1]
    rank = jnp.take_along_axis(ranks_inc, e_all[:, None], axis=1)[:, 0] - 1
    pcounts = ((counts + T - 1) // T) * T
    pcum = jnp.cumsum(pcounts).astype(i32)
    poffs = jnp.concatenate([jnp.zeros(1, i32), pcum[:-1]])
    p_q = poffs[e_all] + rank
    tok = jnp.arange(K * S, dtype=i32) % S
    p1, p2 = p_q[:S], p_q[S:]
    total = pcum[-1]
    tiles = jnp.arange(NT, dtype=i32)
    ar = (tiles * T < total).astype(i32)
    er = jnp.searchsorted(
        pcum, jnp.minimum(tiles * T, total - 1), side="right").astype(i32)
    return tok, p_q, p1, p2, er, ar


PPW = (K * S) // NW
GCH = 16
NGC = PPW // GCH

@functools.lru_cache(maxsize=None)
def _sc_mesh():
    return plsc.VectorSubcoreMesh(core_axis_name="c", subcore_axis_name="s")


@functools.lru_cache(maxsize=None)
def _make_sc_dispatch():
    @functools.partial(
        pl.kernel,
        out_type=jax.ShapeDtypeStruct((P, D), jnp.float32),
        mesh=_sc_mesh(),
        scratch_types=[
            pltpu.VMEM((NGC, GCH), jnp.int32),
            pltpu.VMEM((NGC, GCH), jnp.int32),
            pltpu.VMEM((GCH, D), jnp.float32),
            pltpu.VMEM((GCH, D), jnp.float32),
            pltpu.SemaphoreType.DMA,
            pltpu.SemaphoreType.DMA,
        ],
    )
    def dispatch(x_hbm, tok_hbm, pq_hbm, out_hbm,
                 tok_v, pq_v, buf0, buf1, sem0, sem1):
        wid = lax.axis_index("s") * NC + lax.axis_index("c")
        pltpu.sync_copy(tok_hbm.at[wid], tok_v)
        pltpu.sync_copy(pq_hbm.at[wid], pq_v)
        bufs = (buf0, buf1)
        sems = (sem0, sem1)
        cps = [None, None]
        cps[0] = pltpu.async_copy(x_hbm.at[tok_v.at[0]], buf0, sem0)
        for c in range(NGC):
            if c + 1 < NGC:
                cps[(c + 1) % 2] = pltpu.async_copy(
                    x_hbm.at[tok_v.at[c + 1]], bufs[(c + 1) % 2],
                    sems[(c + 1) % 2])
            cps[c % 2].wait()
            pltpu.sync_copy(bufs[c % 2], out_hbm.at[pq_v.at[c]])

    return dispatch


def _sc_dispatch(xf, tok, p_q):
    tok_r = tok.reshape(NW, NGC, GCH)
    pq_r = p_q.reshape(NW, NGC, GCH)
    return _make_sc_dispatch()(xf, tok_r, pq_r)


TPW = S // NW
CT = 8
NTC = TPW // CT


@functools.lru_cache(maxsize=None)
def _make_sc_combine():
    @functools.partial(
        pl.kernel,
        out_type=jax.ShapeDtypeStruct((S, D), jnp.float32),
        mesh=_sc_mesh(),
        scratch_types=[
            pltpu.VMEM((TPW,), jnp.int32),
            pltpu.VMEM((TPW,), jnp.int32),
            pltpu.VMEM((TPW, L), jnp.float32),
            pltpu.VMEM((TPW, L), jnp.float32),
            pltpu.VMEM((CT, D), jnp.float32),
            pltpu.VMEM((CT, D), jnp.float32),
            pltpu.VMEM((CT, D), jnp.float32),
            pltpu.SemaphoreType.DMA,
            pltpu.SemaphoreType.DMA,
        ],
    )
    def combine(y_hbm, p1_hbm, p2_hbm, g1_hbm, g2_hbm, out_hbm,
                p1_v, p2_v, g1_v, g2_v, y1_v, y2_v, o_v, sem1, sem2):
        wid = lax.axis_index("s") * NC + lax.axis_index("c")
        base = wid * TPW
        pltpu.sync_copy(p1_hbm.at[pl.ds(base, TPW)], p1_v)
        pltpu.sync_copy(p2_hbm.at[pl.ds(base, TPW)], p2_v)
        pltpu.sync_copy(g1_hbm.at[pl.ds(base, TPW)], g1_v)
        pltpu.sync_copy(g2_hbm.at[pl.ds(base, TPW)], g2_v)
        for c in range(NTC):
            cp1 = pltpu.async_copy(y_hbm.at[p1_v.at[pl.ds(c * CT, CT)]], y1_v, sem1)
            cp2 = pltpu.async_copy(y_hbm.at[p2_v.at[pl.ds(c * CT, CT)]], y2_v, sem2)
            cp1.wait()
            cp2.wait()
            for t in range(CT):
                g1 = g1_v[c * CT + t, :]
                g2 = g2_v[c * CT + t, :]

                def body(ci, carry, t=t, g1=g1, g2=g2):
                    sl = pl.ds(ci * L, L)
                    o_v[t, sl] = y1_v[t, sl] * g1 + y2_v[t, sl] * g2
                    return carry

                lax.fori_loop(0, D // L, body, 0)
            pltpu.sync_copy(o_v, out_hbm.at[pl.ds(base + c * CT, CT)])

    return combine


def _sc_combine(y, p1, p2, g1b, g2b):
    return _make_sc_combine()(y, p1, p2, g1b, g2b)


def _ffn_body(er_ref, ar_ref, xg_ref, w1_ref, w2_ref, b1_ref, b2_ref,
              y_ref, acc):
    t = pl.program_id(0)
    j = pl.program_id(1)

    @pl.when(ar_ref[t] > 0)
    def _():
        h = lax.dot_general(xg_ref[...], w1_ref[0], (((1,), (1,)), ((), ())),
                            preferred_element_type=jnp.float32)
        h = jnp.maximum(h + b1_ref[0, 0, 0:1, :], 0.0)
        part = lax.dot_general(h, w2_ref[0], (((1,), (1,)), ((), ())),
                               preferred_element_type=jnp.float32)
        @pl.when(j == 0)
        def _():
            acc[...] = part

        @pl.when(j > 0)
        def _():
            acc[...] = acc[...] + part

        @pl.when(j == NJ - 1)
        def _():
            y_ref[...] = acc[...] + b2_ref[0, 0, 0:1, :]


def _run_ffn(xg, W1, W2, b1, b2, er, ar):
    b1r = jnp.broadcast_to(b1.reshape(E, NJ, 1, BF), (E, NJ, 8, BF))
    b2r = jnp.broadcast_to(b2.reshape(E, 1, 1, D), (E, 1, 8, D))

    y = pl.pallas_call(
        _ffn_body,
        grid_spec=pltpu.PrefetchScalarGridSpec(
            num_scalar_prefetch=2,
            grid=(NT, NJ),
            in_specs=[
                pl.BlockSpec((T, D), lambda t, j, er, ar: (t, 0)),
                pl.BlockSpec((1, BF, D), lambda t, j, er, ar: (er[t], j, 0)),
                pl.BlockSpec((1, D, BF), lambda t, j, er, ar: (er[t], 0, j)),
                pl.BlockSpec((1, 1, 8, BF), lambda t, j, er, ar:
                             (er[t], j, 0, 0)),
                pl.BlockSpec((1, 1, 8, D), lambda t, j, er, ar:
                             (er[t], 0, 0, 0)),
            ],
            out_specs=pl.BlockSpec((T, D), lambda t, j, er, ar: (t, 0)),
            scratch_shapes=[pltpu.VMEM((T, D), jnp.float32)],
        ),
        out_shape=jax.ShapeDtypeStruct((P, D), jnp.float32),
        compiler_params=pltpu.CompilerParams(
            dimension_semantics=("arbitrary", "arbitrary")),
    )(er, ar, xg, W1, W2, b1r, b2r)
    return y


def kernel(x, router_W, router_b, W1, b1, W2, b2):
    orig_shape = x.shape
    xf = x.reshape(-1, D)
    e1, e2, w1, w2 = _run_router(xf, router_W, router_b)
    tok, p_q, p1, p2, er, ar = _routing_metadata(e1, e2)
    xg = _sc_dispatch(xf, tok, p_q)
    y = _run_ffn(xg, W1, W2, b1, b2, er, ar)
    g1b = jnp.broadcast_to(w1[:, None], (S, L))
    g2b = jnp.broadcast_to(w2[:, None], (S, L))
    out = _sc_combine(y, p1, p2, g1b, g2b)
    return out.reshape(orig_shape)


# speedup vs baseline: 1.1178x; 1.1178x over previous
import functools

import jax
import jax.numpy as jnp
from jax import lax
from jax.experimental import pallas as pl
from jax.experimental.pallas import tpu as pltpu
from jax.experimental.pallas import tpu_sc as plsc

E = 8
K = 2
D = 2048
DFF = 8192
S = 2048

T = 512
NT = (K * S + E * (T - 1) + T - 1) // T
P = NT * T
BF = 1024
NJ = DFF // BF
EP = 128
TR = 512

NC = 2
NS = 16
NW = NC * NS
L = 16


def _router_body(x_ref, wt_ref, b_ref, eo_ref, wo_ref):
    x = x_ref[...]
    logits = lax.dot_general(x, wt_ref[...], (((1,), (0,)), ((), ())),
                             preferred_element_type=jnp.float32)
    logits = logits + b_ref[0:1, :]
    lane = lax.broadcasted_iota(jnp.int32, (TR, EP), 1)
    m1 = jnp.max(logits, axis=1, keepdims=True)
    i1 = jnp.min(jnp.where(logits == m1, lane, EP), axis=1, keepdims=True)
    masked = jnp.where(lane == i1, -jnp.inf, logits)
    m2 = jnp.max(masked, axis=1, keepdims=True)
    i2 = jnp.min(jnp.where(masked == m2, lane, EP), axis=1, keepdims=True)
    w1 = 1.0 / (1.0 + jnp.exp(m2 - m1))
    eo_ref[...] = jnp.broadcast_to(i1 * E + i2, (TR, EP)).astype(jnp.int32)
    wo_ref[...] = jnp.broadcast_to(w1, (TR, EP))


def _run_router(xf, router_W, router_b):
    wt = jnp.zeros((D, EP), jnp.float32).at[:, :E].set(router_W.T)
    brow = jnp.full((EP,), -1e30, jnp.float32).at[:E].set(router_b)
    bp = jnp.broadcast_to(brow[None, :], (8, EP))
    eo, wo = pl.pallas_call(
        _router_body,
        grid=(S // TR,),
        in_specs=[
            pl.BlockSpec((TR, D), lambda r: (r, 0)),
            pl.BlockSpec((D, EP), lambda r: (0, 0)),
            pl.BlockSpec((8, EP), lambda r: (0, 0)),
        ],
        out_specs=[
            pl.BlockSpec((TR, EP), lambda r: (r, 0)),
            pl.BlockSpec((TR, EP), lambda r: (r, 0)),
        ],
        out_shape=[
            jax.ShapeDtypeStruct((S, EP), jnp.int32),
            jax.ShapeDtypeStruct((S, EP), jnp.float32),
        ],
    )(xf, wt, bp)
    e12 = eo[:, 0]
    w1 = wo[:, 0]
    return e12 // E, e12 % E, w1, 1.0 - w1


def _routing_metadata(e1, e2):
    i32 = jnp.int32
    e_all = jnp.concatenate([e1, e2]).astype(i32)
    onehot = (e_all[:, None] == jnp.arange(E, dtype=i32)[None, :]).astype(i32)
    ranks_inc = jnp.cumsum(onehot, axis=0)
    counts = ranks_inc[-1]
    rank = jnp.take_along_axis(ranks_inc, e_all[:, None], axis=1)[:, 0] - 1
    pcounts = ((counts + T - 1) // T) * T
    pcum = jnp.cumsum(pcounts).astype(i32)
    poffs = jnp.concatenate([jnp.zeros(1, i32), pcum[:-1]])
    p_q = poffs[e_all] + rank
    tok = jnp.arange(K * S, dtype=i32) % S
    p1, p2 = p_q[:S], p_q[S:]
    total = pcum[-1]
    tiles = jnp.arange(NT, dtype=i32)
    ar = (tiles * T < total).astype(i32)
    er = jnp.searchsorted(
        pcum, jnp.minimum(tiles * T, total - 1), side="right").astype(i32)
    return tok, p_q, p1, p2, er, ar


PPW = (K * S) // NW
GCH = 16
NGC = PPW // GCH

@functools.lru_cache(maxsize=None)
def _sc_mesh():
    return plsc.VectorSubcoreMesh(core_axis_name="c", subcore_axis_name="s")


@functools.lru_cache(maxsize=None)
def _make_sc_dispatch():
    @functools.partial(
        pl.kernel,
        out_type=jax.ShapeDtypeStruct((P, D), jnp.float32),
        mesh=_sc_mesh(),
        scratch_types=[
            pltpu.VMEM((NGC, GCH), jnp.int32),
            pltpu.VMEM((NGC, GCH), jnp.int32),
            pltpu.VMEM((GCH, D), jnp.float32),
            pltpu.VMEM((GCH, D), jnp.float32),
            pltpu.SemaphoreType.DMA,
            pltpu.SemaphoreType.DMA,
        ],
    )
    def dispatch(x_hbm, tok_hbm, pq_hbm, out_hbm,
                 tok_v, pq_v, buf0, buf1, sem0, sem1):
        wid = lax.axis_index("s") * NC + lax.axis_index("c")
        pltpu.sync_copy(tok_hbm.at[wid], tok_v)
        pltpu.sync_copy(pq_hbm.at[wid], pq_v)
        bufs = (buf0, buf1)
        sems = (sem0, sem1)
        cps = [None, None]
        cps[0] = pltpu.async_copy(x_hbm.at[tok_v.at[0]], buf0, sem0)
        for c in range(NGC):
            if c + 1 < NGC:
                cps[(c + 1) % 2] = pltpu.async_copy(
                    x_hbm.at[tok_v.at[c + 1]], bufs[(c + 1) % 2],
                    sems[(c + 1) % 2])
            cps[c % 2].wait()
            pltpu.sync_copy(bufs[c % 2], out_hbm.at[pq_v.at[c]])

    return dispatch


def _sc_dispatch(xf, tok, p_q):
    tok_r = tok.reshape(NW, NGC, GCH)
    pq_r = p_q.reshape(NW, NGC, GCH)
    return _make_sc_dispatch()(xf, tok_r, pq_r)


TPW = S // NW
CT = 8
NTC = TPW // CT


@functools.lru_cache(maxsize=None)
def _make_sc_combine():
    @functools.partial(
        pl.kernel,
        out_type=jax.ShapeDtypeStruct((S, D), jnp.float32),
        mesh=_sc_mesh(),
        scratch_types=[
            pltpu.VMEM((TPW,), jnp.int32),
            pltpu.VMEM((TPW,), jnp.int32),
            pltpu.VMEM((TPW, L), jnp.float32),
            pltpu.VMEM((TPW, L), jnp.float32),
            pltpu.VMEM((CT, D), jnp.float32),
            pltpu.VMEM((CT, D), jnp.float32),
            pltpu.VMEM((CT, D), jnp.float32),
            pltpu.SemaphoreType.DMA,
            pltpu.SemaphoreType.DMA,
        ],
    )
    def combine(y_hbm, p1_hbm, p2_hbm, g1_hbm, g2_hbm, out_hbm,
                p1_v, p2_v, g1_v, g2_v, y1_v, y2_v, o_v, sem1, sem2):
        wid = lax.axis_index("s") * NC + lax.axis_index("c")
        base = wid * TPW
        pltpu.sync_copy(p1_hbm.at[pl.ds(base, TPW)], p1_v)
        pltpu.sync_copy(p2_hbm.at[pl.ds(base, TPW)], p2_v)
        pltpu.sync_copy(g1_hbm.at[pl.ds(base, TPW)], g1_v)
        pltpu.sync_copy(g2_hbm.at[pl.ds(base, TPW)], g2_v)
        for c in range(NTC):
            cp1 = pltpu.async_copy(y_hbm.at[p1_v.at[pl.ds(c * CT, CT)]], y1_v, sem1)
            cp2 = pltpu.async_copy(y_hbm.at[p2_v.at[pl.ds(c * CT, CT)]], y2_v, sem2)
            cp1.wait()
            cp2.wait()
            for t in range(CT):
                g1 = g1_v[c * CT + t, :]
                g2 = g2_v[c * CT + t, :]

                def body(ci, carry, t=t, g1=g1, g2=g2):
                    sl = pl.ds(ci * L, L)
                    o_v[t, sl] = y1_v[t, sl] * g1 + y2_v[t, sl] * g2
                    return carry

                lax.fori_loop(0, D // L, body, 0)
            pltpu.sync_copy(o_v, out_hbm.at[pl.ds(base + c * CT, CT)])

    return combine


def _sc_combine(y, p1, p2, g1b, g2b):
    return _make_sc_combine()(y, p1, p2, g1b, g2b)


def _ffn_body(er_ref, ar_ref, xg_ref, w1_ref, w2_ref, b1_ref, b2_ref,
              y_ref, acc):
    t = pl.program_id(0)
    j = pl.program_id(1)

    @pl.when(ar_ref[t] > 0)
    def _():
        h = lax.dot_general(xg_ref[...], w1_ref[0], (((1,), (1,)), ((), ())),
                            preferred_element_type=jnp.float32)
        h = jnp.maximum(h + b1_ref[0, 0, 0:1, :], 0.0)
        part = lax.dot_general(h, w2_ref[0], (((1,), (1,)), ((), ())),
                               preferred_element_type=jnp.float32)
        @pl.when(j == 0)
        def _():
            acc[...] = part

        @pl.when(j > 0)
        def _():
            acc[...] = acc[...] + part

        @pl.when(j == NJ - 1)
        def _():
            y_ref[...] = acc[...] + b2_ref[0, 0, 0:1, :]


def _run_ffn(xg, W1, W2, b1, b2, er, ar):
    b1r = jnp.broadcast_to(b1.reshape(E, NJ, 1, BF), (E, NJ, 8, BF))
    b2r = jnp.broadcast_to(b2.reshape(E, 1, 1, D), (E, 1, 8, D))

    y = pl.pallas_call(
        _ffn_body,
        grid_spec=pltpu.PrefetchScalarGridSpec(
            num_scalar_prefetch=2,
            grid=(NT, NJ),
            in_specs=[
                pl.BlockSpec((T, D), lambda t, j, er, ar: (t, 0)),
                pl.BlockSpec((1, BF, D), lambda t, j, er, ar: (er[t], j, 0)),
                pl.BlockSpec((1, D, BF), lambda t, j, er, ar: (er[t], 0, j)),
                pl.BlockSpec((1, 1, 8, BF), lambda t, j, er, ar:
                             (er[t], j, 0, 0)),
                pl.BlockSpec((1, 1, 8, D), lambda t, j, er, ar:
                             (er[t], 0, 0, 0)),
            ],
            out_specs=pl.BlockSpec((T, D), lambda t, j, er, ar: (t, 0)),
            scratch_shapes=[pltpu.VMEM((T, D), jnp.float32)],
        ),
        out_shape=jax.ShapeDtypeStruct((P, D), jnp.float32),
        compiler_params=pltpu.CompilerParams(
            dimension_semantics=("arbitrary", "arbitrary")),
    )(er, ar, xg, W1, W2, b1r, b2r)
    return y


def kernel(x, router_W, router_b, W1, b1, W2, b2):
    orig_shape = x.shape
    xf = x.reshape(-1, D)
    e1, e2, w1, w2 = _run_router(xf, router_W, router_b)
    tok, p_q, p1, p2, er, ar = _routing_metadata(e1, e2)
    xg = _sc_dispatch(xf, tok, p_q)
    y = _run_ffn(xg, W1, W2, b1, b2, er, ar)
    g1b = jnp.broadcast_to(w1[:, None], (S, L))
    g2b = jnp.broadcast_to(w2[:, None], (S, L))
    out = _sc_combine(y, p1, p2, g1b, g2b)
    return out.reshape(orig_shape)
